# baseline (speedup 1.0000x reference)
import functools

import jax
import jax.numpy as jnp
from jax.experimental import pallas as pl
from jax.experimental.pallas import tpu as pltpu

DIM = 1024
INNER = 1024
HEADS = 16
DH = INNER // HEADS
L = 2048
QCHUNK = 512


def _matmul_bias_kernel(x_ref, w_ref, b_ref, o_ref):
    acc = jnp.dot(x_ref[...], w_ref[...], preferred_element_type=jnp.float32)
    o_ref[...] = (acc + b_ref[...]).astype(o_ref.dtype)


def _matmul_bias(x2d, w, b, mblk, out_dtype):
    m, k = x2d.shape
    n = w.shape[1]
    return pl.pallas_call(
        _matmul_bias_kernel,
        grid=(m // mblk,),
        in_specs=[
            pl.BlockSpec((mblk, k), lambda i: (i, 0)),
            pl.BlockSpec((k, n), lambda i: (0, 0)),
            pl.BlockSpec((1, n), lambda i: (0, 0)),
        ],
        out_specs=pl.BlockSpec((mblk, n), lambda i: (i, 0)),
        out_shape=jax.ShapeDtypeStruct((m, n), out_dtype),
    )(x2d, w, b.reshape(1, n))


def _attn_kernel(q_ref, k_ref, v_ref, o_ref, *, scale):
    q2 = q_ref[0]
    k2 = k_ref[0]
    v2 = v_ref[0]
    lane = jax.lax.broadcasted_iota(jnp.int32, (1, 2 * DH), 1)
    mask_a = (lane < DH).astype(jnp.bfloat16)
    mask_b = (lane >= DH).astype(jnp.bfloat16)
    out = None
    for mask in (mask_a, mask_b):
        s = jax.lax.dot_general(
            q2 * mask, k2, (((1,), (1,)), ((), ())),
            preferred_element_type=jnp.float32,
        ) * scale
        m = jnp.max(s, axis=-1, keepdims=True)
        p = jnp.exp(s - m)
        l = jnp.sum(p, axis=-1, keepdims=True)
        o = jnp.dot(
            p.astype(jnp.bfloat16), v2, preferred_element_type=jnp.float32
        )
        o = o * (mask.astype(jnp.float32) / l)
        out = o if out is None else out + o
    o_ref[0] = out.astype(o_ref.dtype)


def _attention(qkv, batch):
    npair = HEADS // 2
    grid = (batch, npair, L // QCHUNK)
    scale = DH ** (-0.5)
    return pl.pallas_call(
        functools.partial(_attn_kernel, scale=scale),
        grid=grid,
        in_specs=[
            pl.BlockSpec((1, QCHUNK, 2 * DH), lambda b, j, g: (b, g, j)),
            pl.BlockSpec((1, L, 2 * DH), lambda b, j, g: (b, 0, npair + j)),
            pl.BlockSpec((1, L, 2 * DH), lambda b, j, g: (b, 0, 2 * npair + j)),
        ],
        out_specs=pl.BlockSpec((1, QCHUNK, 2 * DH), lambda b, j, g: (b, g, j)),
        out_shape=jax.ShapeDtypeStruct((batch, L, INNER), jnp.bfloat16),
    )(qkv, qkv, qkv)


def kernel(x, Wqkv, bqkv, Wproj, bproj):
    b, l, d = x.shape
    xb = x.astype(jnp.bfloat16).reshape(b * l, d)
    qkv = _matmul_bias(xb, Wqkv.astype(jnp.bfloat16), bqkv, 512, jnp.bfloat16)
    attn = _attention(qkv.reshape(b, l, 3 * INNER), b)
    out = _matmul_bias(
        attn.reshape(b * l, INNER), Wproj.astype(jnp.bfloat16), bproj, 512,
        jnp.float32,
    )
    return out.reshape(b, l, DIM)


# speedup vs baseline: 1.1531x; 1.1531x over previous
import functools

import jax
import jax.numpy as jnp
from jax.experimental import pallas as pl
from jax.experimental.pallas import tpu as pltpu

DIM = 1024
INNER = 1024
HEADS = 16
DH = INNER // HEADS
L = 2048
QCHUNK = 512


def _matmul_bias_kernel(x_ref, w_ref, b_ref, o_ref):
    acc = jnp.dot(x_ref[...], w_ref[...], preferred_element_type=jnp.float32)
    o_ref[...] = (acc + b_ref[...]).astype(o_ref.dtype)


def _matmul_bias(x2d, w, b, mblk, out_dtype):
    m, k = x2d.shape
    n = w.shape[1]
    return pl.pallas_call(
        _matmul_bias_kernel,
        grid=(m // mblk,),
        in_specs=[
            pl.BlockSpec((mblk, k), lambda i: (i, 0)),
            pl.BlockSpec((k, n), lambda i: (0, 0)),
            pl.BlockSpec((1, n), lambda i: (0, 0)),
        ],
        out_specs=pl.BlockSpec((mblk, n), lambda i: (i, 0)),
        out_shape=jax.ShapeDtypeStruct((m, n), out_dtype),
    )(x2d, w, b.reshape(1, n))


def _attn_kernel(q_ref, k_ref, v_ref, o_ref, *, scale):
    q2 = q_ref[0]
    k2 = k_ref[0]
    v2 = v_ref[0]
    lane = jax.lax.broadcasted_iota(jnp.int32, (1, 2 * DH), 1)
    mask_a = (lane < DH).astype(jnp.float32)
    mask_b = (lane >= DH).astype(jnp.float32)
    c = jnp.float32(scale * 1.4426950408889634)
    qs = q2.astype(jnp.float32) * c
    v_aug = jnp.concatenate(
        [v2, jnp.ones((v2.shape[0], 2 * DH), jnp.bfloat16)], axis=1
    )
    out = None
    for mask in (mask_a, mask_b):
        s = jax.lax.dot_general(
            (qs * mask).astype(jnp.bfloat16), k2, (((1,), (1,)), ((), ())),
            preferred_element_type=jnp.float32,
        )
        m = jnp.max(s, axis=-1, keepdims=True)
        p16 = jnp.exp2(s - m).astype(jnp.bfloat16)
        o_aug = jnp.dot(
            p16, v_aug, preferred_element_type=jnp.float32
        )
        o = o_aug[:, : 2 * DH] * (mask / o_aug[:, 2 * DH :])
        out = o if out is None else out + o
    o_ref[0] = out.astype(o_ref.dtype)


def _attention(qkv, batch):
    npair = HEADS // 2
    grid = (batch, npair, L // QCHUNK)
    scale = DH ** (-0.5)
    return pl.pallas_call(
        functools.partial(_attn_kernel, scale=scale),
        grid=grid,
        in_specs=[
            pl.BlockSpec((1, QCHUNK, 2 * DH), lambda b, j, g: (b, g, j)),
            pl.BlockSpec((1, L, 2 * DH), lambda b, j, g: (b, 0, npair + j)),
            pl.BlockSpec((1, L, 2 * DH), lambda b, j, g: (b, 0, 2 * npair + j)),
        ],
        out_specs=pl.BlockSpec((1, QCHUNK, 2 * DH), lambda b, j, g: (b, g, j)),
        out_shape=jax.ShapeDtypeStruct((batch, L, INNER), jnp.bfloat16),
    )(qkv, qkv, qkv)


def kernel(x, Wqkv, bqkv, Wproj, bproj):
    b, l, d = x.shape
    xb = x.astype(jnp.bfloat16).reshape(b * l, d)
    qkv = _matmul_bias(xb, Wqkv.astype(jnp.bfloat16), bqkv, 512, jnp.bfloat16)
    attn = _attention(qkv.reshape(b, l, 3 * INNER), b)
    out = _matmul_bias(
        attn.reshape(b * l, INNER), Wproj.astype(jnp.bfloat16), bproj, 512,
        jnp.float32,
    )
    return out.reshape(b, l, DIM)
